# baseline (speedup 1.0000x reference)
import functools

import jax
import jax.numpy as jnp
from jax import lax
from jax.experimental import pallas as pl
from jax.experimental.pallas import tpu as pltpu
from jax.experimental.pallas import tpu_sc as plsc

_INFO = plsc.get_sparse_core_info()
_NC = _INFO.num_cores
_NS = _INFO.num_subcores
_NW = _NC * _NS
_CHUNK = 128
_T = 512


def _pack_body(ugT, umT, igT, imT, upack, ipack):
    h = _T // 2

    def pack(aT, bT):
        a = aT[...].T
        b = bT[...].T
        return jnp.concatenate([a[:h], b[:h], a[h:], b[h:]], axis=1)

    upack[...] = pack(ugT, umT)
    ipack[...] = pack(igT, imT)


@functools.partial(jax.jit, static_argnames=())
def _tc_pack(ugT, umT, igT, imT):
    d, v = ugT.shape
    g = (v + _T - 1) // _T
    v2 = g * (_T // 2)
    in_spec = pl.BlockSpec((d, _T), lambda i: (0, i))
    out_spec = pl.BlockSpec((_T // 2, 4 * d), lambda i: (i, 0))
    out_sh = jax.ShapeDtypeStruct((v2, 4 * d), jnp.float32)
    return pl.pallas_call(
        _pack_body,
        grid=(g,),
        in_specs=[in_spec] * 4,
        out_specs=(out_spec, out_spec),
        out_shape=(out_sh, out_sh),
    )(ugT, umT, igT, imT)


def _sc_gather_body(uidx_hbm, iidx_hbm, upack, ipack, uout, iout,
                    idx_v, rows_v, sem, *, rows_per_w, bpw):
    wid = lax.axis_index("s") * _NC + lax.axis_index("c")
    rbase = wid * rows_per_w
    base = wid * bpw
    for (src_idx, tab, out) in ((uidx_hbm, upack, uout),
                                (iidx_hbm, ipack, iout)):
        pltpu.sync_copy(src_idx.at[pl.ds(rbase, rows_per_w)], idx_v)
        handles = []
        for j in range(rows_per_w):
            sl = pl.ds(j * _CHUNK, _CHUNK)
            handles.append(
                pltpu.async_copy(tab.at[idx_v.at[j]], rows_v.at[sl], sem))
        for h in handles:
            h.wait()
        pltpu.sync_copy(rows_v, out.at[pl.ds(base, bpw)])


@functools.partial(jax.jit, static_argnames=("b",))
def _sc_gather(uidx2d, iidx2d, upack, ipack, *, b):
    bpw = b // _NW
    rows_per_w = bpw // _CHUNK
    w = upack.shape[1]
    mesh = plsc.VectorSubcoreMesh(core_axis_name="c", subcore_axis_name="s")
    out_sh = jax.ShapeDtypeStruct((b, w), jnp.float32)
    f = pl.kernel(
        functools.partial(_sc_gather_body, rows_per_w=rows_per_w, bpw=bpw),
        mesh=mesh,
        out_type=(out_sh, out_sh),
        scratch_types=[
            pltpu.VMEM((rows_per_w, _CHUNK), jnp.int32),
            pltpu.VMEM((bpw, w), jnp.float32),
            pltpu.SemaphoreType.DMA,
        ],
    )
    return f(uidx2d, iidx2d, upack, ipack)


def _tc_mlp_body(urows, irows, upar, ipar, w1a, w1b, b1, w2, b2,
                 wfa, wfb, bf, out):
    d = w1a.shape[0]
    usel = jnp.where(upar[...] > 0, urows[:, 2 * d:], urows[:, :2 * d])
    isel = jnp.where(ipar[...] > 0, irows[:, 2 * d:], irows[:, :2 * d])
    ug, um = usel[:, :d], usel[:, d:]
    ig, im = isel[:, :d], isel[:, d:]
    h = um @ w1a[...] + im @ w1b[...] + b1[...]
    h = jnp.maximum(h, 0.0)
    h = jnp.maximum(h @ w2[...] + b2[...], 0.0)
    out[...] = (ug * ig) @ wfa[...] + h @ wfb[...] + bf[...]


@functools.partial(jax.jit, static_argnames=("blk",))
def _tc_mlp(urows, irows, upar, ipar, w1a, w1b, b1, w2, b2, wfa, wfb, bf,
            *, blk):
    b, w = urows.shape
    grid = (b // blk,)
    row_spec = pl.BlockSpec((blk, w), lambda i: (i, 0))
    par_spec = pl.BlockSpec((blk, 1), lambda i: (i, 0))
    full = lambda a: pl.BlockSpec(a.shape, lambda i: (0,) * a.ndim)
    return pl.pallas_call(
        _tc_mlp_body,
        grid=grid,
        in_specs=[row_spec, row_spec, par_spec, par_spec,
                  full(w1a), full(w1b), full(b1), full(w2), full(b2),
                  full(wfa), full(wfb), full(bf)],
        out_specs=pl.BlockSpec((blk, 1), lambda i: (i, 0)),
        out_shape=jax.ShapeDtypeStruct((b, 1), jnp.float32),
    )(urows, irows, upar, ipar, w1a, w1b, b1, w2, b2, wfa, wfb, bf)


def kernel(inputs, user_emb_gmf, item_emb_gmf, user_emb_mlp, item_emb_mlp,
           W1, b1, W2, b2, Wf, bf):
    b = inputs.shape[0]
    d = user_emb_gmf.shape[1]
    h = _T // 2
    uid = inputs[:, 0]
    iid = inputs[:, 1]
    upack, ipack = _tc_pack(user_emb_gmf.T, user_emb_mlp.T,
                            item_emb_gmf.T, item_emb_mlp.T)
    urow = (uid // _T) * h + uid % h
    irow = (iid // _T) * h + iid % h
    uhalf = (uid // h) % 2
    ihalf = (iid // h) % 2
    urows, irows = _sc_gather(urow.reshape(-1, _CHUNK),
                              irow.reshape(-1, _CHUNK), upack, ipack, b=b)
    out = _tc_mlp(urows, irows,
                  uhalf.reshape(-1, 1), ihalf.reshape(-1, 1),
                  W1[:d], W1[d:], b1.reshape(1, -1),
                  W2, b2.reshape(1, -1),
                  Wf[:d], Wf[d:], bf.reshape(1, 1),
                  blk=2048)
    return out


# speedup vs baseline: 1.0927x; 1.0927x over previous
import functools

import jax
import jax.numpy as jnp
from jax import lax
from jax.experimental import pallas as pl
from jax.experimental.pallas import tpu as pltpu
from jax.experimental.pallas import tpu_sc as plsc

_INFO = plsc.get_sparse_core_info()
_NC = _INFO.num_cores
_NS = _INFO.num_subcores
_NW = _NC * _NS
_CHUNK = 128
_T = 512


def _pack_body(ugT, umT, igT, imT, eye, upack, ipack):
    h = _T // 2
    d = ugT.shape[0]
    x = jnp.concatenate([ugT[...], umT[...], igT[...], imT[...]], axis=0)
    xt = lax.dot_general(x, eye[...], (((0,), (0,)), ((), ())),
                         preferred_element_type=jnp.float32)
    upack[...] = jnp.concatenate(
        [xt[:h, :d], xt[:h, d:2 * d], xt[h:, :d], xt[h:, d:2 * d]], axis=1)
    ipack[...] = jnp.concatenate(
        [xt[:h, 2 * d:3 * d], xt[:h, 3 * d:], xt[h:, 2 * d:3 * d],
         xt[h:, 3 * d:]], axis=1)


@functools.partial(jax.jit, static_argnames=())
def _tc_pack(ugT, umT, igT, imT):
    d, v = ugT.shape
    g = (v + _T - 1) // _T
    v2 = g * (_T // 2)
    eye = jnp.eye(4 * d, dtype=jnp.float32)
    in_spec = pl.BlockSpec((d, _T), lambda i: (0, i))
    eye_spec = pl.BlockSpec((4 * d, 4 * d), lambda i: (0, 0))
    out_spec = pl.BlockSpec((_T // 2, 4 * d), lambda i: (i, 0))
    out_sh = jax.ShapeDtypeStruct((v2, 4 * d), jnp.float32)
    return pl.pallas_call(
        _pack_body,
        grid=(g,),
        in_specs=[in_spec] * 4 + [eye_spec],
        out_specs=(out_spec, out_spec),
        out_shape=(out_sh, out_sh),
    )(ugT, umT, igT, imT, eye)


def _sc_gather_body(uidx_hbm, iidx_hbm, upack, ipack, uout, iout,
                    idx_v, rows_v, sem, *, rows_per_w, bpw):
    wid = lax.axis_index("s") * _NC + lax.axis_index("c")
    rbase = wid * rows_per_w
    base = wid * bpw
    for (src_idx, tab, out) in ((uidx_hbm, upack, uout),
                                (iidx_hbm, ipack, iout)):
        pltpu.sync_copy(src_idx.at[pl.ds(rbase, rows_per_w)], idx_v)
        handles = []
        for j in range(rows_per_w):
            sl = pl.ds(j * _CHUNK, _CHUNK)
            handles.append(
                pltpu.async_copy(tab.at[idx_v.at[j]], rows_v.at[sl], sem))
        for h in handles:
            h.wait()
        pltpu.sync_copy(rows_v, out.at[pl.ds(base, bpw)])


@functools.partial(jax.jit, static_argnames=("b",))
def _sc_gather(uidx2d, iidx2d, upack, ipack, *, b):
    bpw = b // _NW
    rows_per_w = bpw // _CHUNK
    w = upack.shape[1]
    mesh = plsc.VectorSubcoreMesh(core_axis_name="c", subcore_axis_name="s")
    out_sh = jax.ShapeDtypeStruct((b, w), jnp.float32)
    f = pl.kernel(
        functools.partial(_sc_gather_body, rows_per_w=rows_per_w, bpw=bpw),
        mesh=mesh,
        out_type=(out_sh, out_sh),
        scratch_types=[
            pltpu.VMEM((rows_per_w, _CHUNK), jnp.int32),
            pltpu.VMEM((bpw, w), jnp.float32),
            pltpu.SemaphoreType.DMA,
        ],
    )
    return f(uidx2d, iidx2d, upack, ipack)


def _tc_mlp_body(urows, irows, upar, ipar, w1a, w1b, b1, w2, b2,
                 wfa, wfb, bf, out):
    d = w1a.shape[0]
    usel = jnp.where(upar[...] > 0, urows[:, 2 * d:], urows[:, :2 * d])
    isel = jnp.where(ipar[...] > 0, irows[:, 2 * d:], irows[:, :2 * d])
    ug, um = usel[:, :d], usel[:, d:]
    ig, im = isel[:, :d], isel[:, d:]
    h = um @ w1a[...] + im @ w1b[...] + b1[...]
    h = jnp.maximum(h, 0.0)
    h = jnp.maximum(h @ w2[...] + b2[...], 0.0)
    out[...] = (ug * ig) @ wfa[...] + h @ wfb[...] + bf[...]


@functools.partial(jax.jit, static_argnames=("blk",))
def _tc_mlp(urows, irows, upar, ipar, w1a, w1b, b1, w2, b2, wfa, wfb, bf,
            *, blk):
    b, w = urows.shape
    grid = (b // blk,)
    row_spec = pl.BlockSpec((blk, w), lambda i: (i, 0))
    par_spec = pl.BlockSpec((blk, 1), lambda i: (i, 0))
    full = lambda a: pl.BlockSpec(a.shape, lambda i: (0,) * a.ndim)
    return pl.pallas_call(
        _tc_mlp_body,
        grid=grid,
        in_specs=[row_spec, row_spec, par_spec, par_spec,
                  full(w1a), full(w1b), full(b1), full(w2), full(b2),
                  full(wfa), full(wfb), full(bf)],
        out_specs=pl.BlockSpec((blk, 1), lambda i: (i, 0)),
        out_shape=jax.ShapeDtypeStruct((b, 1), jnp.float32),
    )(urows, irows, upar, ipar, w1a, w1b, b1, w2, b2, wfa, wfb, bf)


def kernel(inputs, user_emb_gmf, item_emb_gmf, user_emb_mlp, item_emb_mlp,
           W1, b1, W2, b2, Wf, bf):
    b = inputs.shape[0]
    d = user_emb_gmf.shape[1]
    h = _T // 2
    uid = inputs[:, 0]
    iid = inputs[:, 1]
    upack, ipack = _tc_pack(user_emb_gmf.T, user_emb_mlp.T,
                            item_emb_gmf.T, item_emb_mlp.T)
    urow = (uid // _T) * h + uid % h
    irow = (iid // _T) * h + iid % h
    uhalf = (uid // h) % 2
    ihalf = (iid // h) % 2
    urows, irows = _sc_gather(urow.reshape(-1, _CHUNK),
                              irow.reshape(-1, _CHUNK), upack, ipack, b=b)
    out = _tc_mlp(urows, irows,
                  uhalf.reshape(-1, 1), ihalf.reshape(-1, 1),
                  W1[:d], W1[d:], b1.reshape(1, -1),
                  W2, b2.reshape(1, -1),
                  Wf[:d], Wf[d:], bf.reshape(1, 1),
                  blk=2048)
    return out


# speedup vs baseline: 1.2076x; 1.1052x over previous
import functools

import jax
import jax.numpy as jnp
from jax import lax
from jax.experimental import pallas as pl
from jax.experimental.pallas import tpu as pltpu
from jax.experimental.pallas import tpu_sc as plsc

_INFO = plsc.get_sparse_core_info()
_NC = _INFO.num_cores
_NS = _INFO.num_subcores
_NW = _NC * _NS
_CHUNK = 128
_T = 512


def _pack_body(ugT, umT, igT, imT, eye, upack, ipack):
    d = ugT.shape[0]
    x = jnp.concatenate([ugT[...], umT[...], igT[...], imT[...]], axis=0)
    xt = lax.dot_general(x, eye[...], (((0,), (0,)), ((), ())),
                         preferred_element_type=jnp.float32)
    xu = xt[:, :2 * d]
    xi = xt[:, 2 * d:]
    upack[...] = jnp.concatenate([xu, xu], axis=1)
    ipack[...] = jnp.concatenate([xi, xi], axis=1)


@functools.partial(jax.jit, static_argnames=())
def _tc_pack(ugT, umT, igT, imT):
    d, v = ugT.shape
    g = (v + _T - 1) // _T
    vp = g * _T
    eye = jnp.eye(4 * d, dtype=jnp.float32)
    in_spec = pl.BlockSpec((d, _T), lambda i: (0, i))
    eye_spec = pl.BlockSpec((4 * d, 4 * d), lambda i: (0, 0))
    out_spec = pl.BlockSpec((_T, 4 * d), lambda i: (i, 0))
    out_sh = jax.ShapeDtypeStruct((vp, 4 * d), jnp.float32)
    return pl.pallas_call(
        _pack_body,
        grid=(g,),
        in_specs=[in_spec] * 4 + [eye_spec],
        out_specs=(out_spec, out_spec),
        out_shape=(out_sh, out_sh),
    )(ugT, umT, igT, imT, eye)


def _sc_gather_body(uid_hbm, iid_hbm, upack, ipack, uout, iout,
                    ids_v, rows_v, sem, *, rows_per_w, bpw):
    wid = lax.axis_index("s") * _NC + lax.axis_index("c")
    rbase = wid * rows_per_w
    base = wid * bpw
    for (src_ids, tab, out) in ((uid_hbm, upack, uout),
                                (iid_hbm, ipack, iout)):
        pltpu.sync_copy(src_ids.at[pl.ds(rbase, rows_per_w)], ids_v)
        handles = []
        for j in range(rows_per_w):
            sl = pl.ds(j * _CHUNK, _CHUNK)
            handles.append(
                pltpu.async_copy(tab.at[ids_v.at[j]], rows_v.at[sl], sem))
        for h in handles:
            h.wait()
        pltpu.sync_copy(rows_v, out.at[pl.ds(base, bpw)])


@functools.partial(jax.jit, static_argnames=("b",))
def _sc_gather(uid2d, iid2d, upack, ipack, *, b):
    bpw = b // _NW
    rows_per_w = bpw // _CHUNK
    w = upack.shape[1]
    mesh = plsc.VectorSubcoreMesh(core_axis_name="c", subcore_axis_name="s")
    out_sh = jax.ShapeDtypeStruct((b, w), jnp.float32)
    f = pl.kernel(
        functools.partial(_sc_gather_body, rows_per_w=rows_per_w, bpw=bpw),
        mesh=mesh,
        out_type=(out_sh, out_sh),
        scratch_types=[
            pltpu.VMEM((rows_per_w, _CHUNK), jnp.int32),
            pltpu.VMEM((bpw, w), jnp.float32),
            pltpu.SemaphoreType.DMA,
        ],
    )
    return f(uid2d, iid2d, upack, ipack)


def _tc_mlp_body(usel, isel, w1a, w1b, b1, w2, b2, wfa, wfb, bf, out):
    d = w1a.shape[0]
    ug, um = usel[:, :d], usel[:, d:2 * d]
    ig, im = isel[:, :d], isel[:, d:2 * d]
    h = um @ w1a[...] + im @ w1b[...] + b1[...]
    h = jnp.maximum(h, 0.0)
    h = jnp.maximum(h @ w2[...] + b2[...], 0.0)
    out[...] = (ug * ig) @ wfa[...] + h @ wfb[...] + bf[...]


@functools.partial(jax.jit, static_argnames=("blk",))
def _tc_mlp(usel, isel, w1a, w1b, b1, w2, b2, wfa, wfb, bf, *, blk):
    b, w = usel.shape
    grid = (b // blk,)
    row_spec = pl.BlockSpec((blk, w), lambda i: (i, 0))
    full = lambda a: pl.BlockSpec(a.shape, lambda i: (0,) * a.ndim)
    return pl.pallas_call(
        _tc_mlp_body,
        grid=grid,
        in_specs=[row_spec, row_spec,
                  full(w1a), full(w1b), full(b1), full(w2), full(b2),
                  full(wfa), full(wfb), full(bf)],
        out_specs=pl.BlockSpec((blk, 1), lambda i: (i, 0)),
        out_shape=jax.ShapeDtypeStruct((b, 1), jnp.float32),
    )(usel, isel, w1a, w1b, b1, w2, b2, wfa, wfb, bf)


def kernel(inputs, user_emb_gmf, item_emb_gmf, user_emb_mlp, item_emb_mlp,
           W1, b1, W2, b2, Wf, bf):
    b = inputs.shape[0]
    d = user_emb_gmf.shape[1]
    uid2d = inputs[:, 0].reshape(-1, _CHUNK)
    iid2d = inputs[:, 1].reshape(-1, _CHUNK)
    upack, ipack = _tc_pack(user_emb_gmf.T, user_emb_mlp.T,
                            item_emb_gmf.T, item_emb_mlp.T)
    usel, isel = _sc_gather(uid2d, iid2d, upack, ipack, b=b)
    out = _tc_mlp(usel, isel,
                  W1[:d], W1[d:], b1.reshape(1, -1),
                  W2, b2.reshape(1, -1),
                  Wf[:d], Wf[d:], bf.reshape(1, 1),
                  blk=4096)
    return out


# speedup vs baseline: 1.7033x; 1.4105x over previous
import functools

import jax
import jax.numpy as jnp
from jax import lax
from jax.experimental import pallas as pl
from jax.experimental.pallas import tpu as pltpu
from jax.experimental.pallas import tpu_sc as plsc

_INFO = plsc.get_sparse_core_info()
_NC = _INFO.num_cores
_NS = _INFO.num_subcores
_NW = _NC * _NS
_CHUNK = 128
_T = 2048


def _pack_body(ugT, umT, igT, imT, eye, w1a, w1b, upack, ipack):
    d = ugT.shape[0]
    t = ugT.shape[1]
    dims = (((0,), (0,)), ((), ()))
    ugt = lax.dot_general(ugT[...], eye[...], dims,
                          preferred_element_type=jnp.float32)
    igt = lax.dot_general(igT[...], eye[...], dims,
                          preferred_element_type=jnp.float32)
    pu = lax.dot_general(umT[...], w1a[...], dims,
                         preferred_element_type=jnp.float32)
    pi = lax.dot_general(imT[...], w1b[...], dims,
                         preferred_element_type=jnp.float32)
    z = jnp.zeros((t, d), jnp.float32)
    upack[...] = jnp.concatenate([ugt, pu, z], axis=1)
    ipack[...] = jnp.concatenate([igt, pi, z], axis=1)


@functools.partial(jax.jit, static_argnames=())
def _tc_pack(ugT, umT, igT, imT, w1a, w1b):
    d, v = ugT.shape
    g = (v + _T - 1) // _T
    vp = g * _T
    eye = jnp.eye(d, dtype=jnp.float32)
    in_spec = pl.BlockSpec((d, _T), lambda i: (0, i))
    full = lambda a: pl.BlockSpec(a.shape, lambda i: (0,) * a.ndim)
    out_spec = pl.BlockSpec((_T, 4 * d), lambda i: (i, 0))
    out_sh = jax.ShapeDtypeStruct((vp, 4 * d), jnp.float32)
    return pl.pallas_call(
        _pack_body,
        grid=(g,),
        in_specs=[in_spec] * 4 + [full(eye), full(w1a), full(w1b)],
        out_specs=(out_spec, out_spec),
        out_shape=(out_sh, out_sh),
    )(ugT, umT, igT, imT, eye, w1a, w1b)


def _sc_gather_body(uid_hbm, iid_hbm, upack, ipack, uout, iout,
                    ids_v, rows_v, sem, *, rows_per_w, bpw):
    wid = lax.axis_index("s") * _NC + lax.axis_index("c")
    rbase = wid * rows_per_w
    base = wid * bpw
    for (src_ids, tab, out) in ((uid_hbm, upack, uout),
                                (iid_hbm, ipack, iout)):
        pltpu.sync_copy(src_ids.at[pl.ds(rbase, rows_per_w)], ids_v)
        handles = []
        for j in range(rows_per_w):
            sl = pl.ds(j * _CHUNK, _CHUNK)
            handles.append(
                pltpu.async_copy(tab.at[ids_v.at[j]], rows_v.at[sl], sem))
        for h in handles:
            h.wait()
        pltpu.sync_copy(rows_v, out.at[pl.ds(base, bpw)])


@functools.partial(jax.jit, static_argnames=("b",))
def _sc_gather(uid2d, iid2d, upack, ipack, *, b):
    bpw = b // _NW
    rows_per_w = bpw // _CHUNK
    w = upack.shape[1]
    mesh = plsc.VectorSubcoreMesh(core_axis_name="c", subcore_axis_name="s")
    out_sh = jax.ShapeDtypeStruct((b, w), jnp.float32)
    f = pl.kernel(
        functools.partial(_sc_gather_body, rows_per_w=rows_per_w, bpw=bpw),
        mesh=mesh,
        out_type=(out_sh, out_sh),
        scratch_types=[
            pltpu.VMEM((rows_per_w, _CHUNK), jnp.int32),
            pltpu.VMEM((bpw, w), jnp.float32),
            pltpu.SemaphoreType.DMA,
        ],
    )
    return f(uid2d, iid2d, upack, ipack)


def _tc_mlp_body(usel, isel, b1, w2, b2, wfa, wfb, bf, out):
    d = wfa.shape[0]
    ug = usel[:, :d]
    ig = isel[:, :d]
    h = usel[:, d:3 * d] + isel[:, d:3 * d] + b1[...]
    h = jnp.maximum(h, 0.0)
    h = jnp.maximum(h @ w2[...] + b2[...], 0.0)
    out[...] = (ug * ig) @ wfa[...] + h @ wfb[...] + bf[...]


@functools.partial(jax.jit, static_argnames=("blk",))
def _tc_mlp(usel, isel, b1, w2, b2, wfa, wfb, bf, *, blk):
    b, w = usel.shape
    grid = (b // blk,)
    row_spec = pl.BlockSpec((blk, w), lambda i: (i, 0))
    full = lambda a: pl.BlockSpec(a.shape, lambda i: (0,) * a.ndim)
    return pl.pallas_call(
        _tc_mlp_body,
        grid=grid,
        in_specs=[row_spec, row_spec,
                  full(b1), full(w2), full(b2),
                  full(wfa), full(wfb), full(bf)],
        out_specs=pl.BlockSpec((blk, 1), lambda i: (i, 0)),
        out_shape=jax.ShapeDtypeStruct((b, 1), jnp.float32),
    )(usel, isel, b1, w2, b2, wfa, wfb, bf)


def kernel(inputs, user_emb_gmf, item_emb_gmf, user_emb_mlp, item_emb_mlp,
           W1, b1, W2, b2, Wf, bf):
    b = inputs.shape[0]
    d = user_emb_gmf.shape[1]
    uid2d = inputs[:, 0].reshape(-1, _CHUNK)
    iid2d = inputs[:, 1].reshape(-1, _CHUNK)
    upack, ipack = _tc_pack(user_emb_gmf.T, user_emb_mlp.T,
                            item_emb_gmf.T, item_emb_mlp.T,
                            W1[:d], W1[d:])
    usel, isel = _sc_gather(uid2d, iid2d, upack, ipack, b=b)
    out = _tc_mlp(usel, isel,
                  b1.reshape(1, -1),
                  W2, b2.reshape(1, -1),
                  Wf[:d], Wf[d:], bf.reshape(1, 1),
                  blk=4096)
    return out


# speedup vs baseline: 2.0359x; 1.1952x over previous
import functools

import jax
import jax.numpy as jnp
from jax import lax
from jax.experimental import pallas as pl
from jax.experimental.pallas import tpu as pltpu
from jax.experimental.pallas import tpu_sc as plsc

_INFO = plsc.get_sparse_core_info()
_NC = _INFO.num_cores
_NS = _INFO.num_subcores
_NW = _NC * _NS
_CHUNK = 128
_T = 2048


def _pack_body(ugT, umT, igT, imT, ru, ri, upack, ipack):
    dims = (((0,), (0,)), ((), ()))
    xu = jnp.concatenate([ugT[...], umT[...]], axis=0)
    xi = jnp.concatenate([igT[...], imT[...]], axis=0)
    upack[...] = lax.dot_general(xu, ru[...], dims,
                                 preferred_element_type=jnp.float32)
    ipack[...] = lax.dot_general(xi, ri[...], dims,
                                 preferred_element_type=jnp.float32)


@functools.partial(jax.jit, static_argnames=())
def _tc_pack(ugT, umT, igT, imT, w1a, w1b):
    d, v = ugT.shape
    g = (v + _T - 1) // _T
    vp = g * _T
    eye = jnp.eye(d, dtype=jnp.float32)
    z = jnp.zeros((d, d), jnp.float32)
    zwide = jnp.zeros((d, 2 * d), jnp.float32)
    ru = jnp.block([[eye, zwide, z], [z, w1a, z]])
    ri = jnp.block([[eye, zwide, z], [z, w1b, z]])
    in_spec = pl.BlockSpec((d, _T), lambda i: (0, i))
    full = lambda a: pl.BlockSpec(a.shape, lambda i: (0,) * a.ndim)
    out_spec = pl.BlockSpec((_T, 4 * d), lambda i: (i, 0))
    out_sh = jax.ShapeDtypeStruct((vp, 4 * d), jnp.float32)
    return pl.pallas_call(
        _pack_body,
        grid=(g,),
        in_specs=[in_spec] * 4 + [full(ru), full(ri)],
        out_specs=(out_spec, out_spec),
        out_shape=(out_sh, out_sh),
        compiler_params=pltpu.CompilerParams(
            fuse_transposed_lhs_in_matmul=True),
    )(ugT, umT, igT, imT, ru, ri)


def _sc_gather_body(uid_hbm, iid_hbm, upack, ipack, uout, iout,
                    ids_v, rows_v, sem, *, rows_per_w, bpw):
    wid = lax.axis_index("s") * _NC + lax.axis_index("c")
    rbase = wid * rows_per_w
    base = wid * bpw
    for (src_ids, tab, out) in ((uid_hbm, upack, uout),
                                (iid_hbm, ipack, iout)):
        pltpu.sync_copy(src_ids.at[pl.ds(rbase, rows_per_w)], ids_v)
        handles = []
        for j in range(rows_per_w):
            sl = pl.ds(j * _CHUNK, _CHUNK)
            handles.append(
                pltpu.async_copy(tab.at[ids_v.at[j]], rows_v.at[sl], sem))
        for h in handles:
            h.wait()
        pltpu.sync_copy(rows_v, out.at[pl.ds(base, bpw)])


@functools.partial(jax.jit, static_argnames=("b",))
def _sc_gather(uid2d, iid2d, upack, ipack, *, b):
    bpw = b // _NW
    rows_per_w = bpw // _CHUNK
    w = upack.shape[1]
    mesh = plsc.VectorSubcoreMesh(core_axis_name="c", subcore_axis_name="s")
    out_sh = jax.ShapeDtypeStruct((b, w), jnp.float32)
    f = pl.kernel(
        functools.partial(_sc_gather_body, rows_per_w=rows_per_w, bpw=bpw),
        mesh=mesh,
        out_type=(out_sh, out_sh),
        scratch_types=[
            pltpu.VMEM((rows_per_w, _CHUNK), jnp.int32),
            pltpu.VMEM((bpw, w), jnp.float32),
            pltpu.SemaphoreType.DMA,
        ],
    )
    return f(uid2d, iid2d, upack, ipack)


def _tc_mlp_body(usel, isel, b1, w2, b2, wfa, wfb, bf, out):
    d = wfa.shape[0]
    ug = usel[:, :d]
    ig = isel[:, :d]
    h = usel[:, d:3 * d] + isel[:, d:3 * d] + b1[...]
    h = jnp.maximum(h, 0.0)
    h = jnp.maximum(h @ w2[...] + b2[...], 0.0)
    out[...] = (ug * ig) @ wfa[...] + h @ wfb[...] + bf[...]


@functools.partial(jax.jit, static_argnames=("blk",))
def _tc_mlp(usel, isel, b1, w2, b2, wfa, wfb, bf, *, blk):
    b, w = usel.shape
    grid = (b // blk,)
    row_spec = pl.BlockSpec((blk, w), lambda i: (i, 0))
    full = lambda a: pl.BlockSpec(a.shape, lambda i: (0,) * a.ndim)
    return pl.pallas_call(
        _tc_mlp_body,
        grid=grid,
        in_specs=[row_spec, row_spec,
                  full(b1), full(w2), full(b2),
                  full(wfa), full(wfb), full(bf)],
        out_specs=pl.BlockSpec((blk, 1), lambda i: (i, 0)),
        out_shape=jax.ShapeDtypeStruct((b, 1), jnp.float32),
    )(usel, isel, b1, w2, b2, wfa, wfb, bf)


def kernel(inputs, user_emb_gmf, item_emb_gmf, user_emb_mlp, item_emb_mlp,
           W1, b1, W2, b2, Wf, bf):
    b = inputs.shape[0]
    d = user_emb_gmf.shape[1]
    uid2d = inputs[:, 0].reshape(-1, _CHUNK)
    iid2d = inputs[:, 1].reshape(-1, _CHUNK)
    upack, ipack = _tc_pack(user_emb_gmf.T, user_emb_mlp.T,
                            item_emb_gmf.T, item_emb_mlp.T,
                            W1[:d], W1[d:])
    usel, isel = _sc_gather(uid2d, iid2d, upack, ipack, b=b)
    out = _tc_mlp(usel, isel,
                  b1.reshape(1, -1),
                  W2, b2.reshape(1, -1),
                  Wf[:d], Wf[d:], bf.reshape(1, 1),
                  blk=4096)
    return out


# speedup vs baseline: 2.3271x; 1.1430x over previous
import functools

import jax
import jax.numpy as jnp
from jax import lax
from jax.experimental import pallas as pl
from jax.experimental.pallas import tpu as pltpu
from jax.experimental.pallas import tpu_sc as plsc

_INFO = plsc.get_sparse_core_info()
_NC = _INFO.num_cores
_NS = _INFO.num_subcores
_NW = _NC * _NS
_CHUNK = 128
_T = 4096


def _pack_body(ugT, umT, igT, imT, ru, ri, upack, ipack):
    dims = (((0,), (0,)), ((), ()))
    xu = jnp.concatenate([ugT[...], umT[...]], axis=0)
    xi = jnp.concatenate([igT[...], imT[...]], axis=0)
    upack[...] = lax.dot_general(xu, ru[...], dims,
                                 preferred_element_type=jnp.float32)
    ipack[...] = lax.dot_general(xi, ri[...], dims,
                                 preferred_element_type=jnp.float32)


@functools.partial(jax.jit, static_argnames=())
def _tc_pack(ugT, umT, igT, imT, w1a, w1b):
    d, v = ugT.shape
    g = (v + _T - 1) // _T
    vp = g * _T
    eye = jnp.eye(d, dtype=jnp.float32)
    z = jnp.zeros((d, d), jnp.float32)
    zwide = jnp.zeros((d, 2 * d), jnp.float32)
    ru = jnp.block([[eye, zwide, z], [z, w1a, z]])
    ri = jnp.block([[eye, zwide, z], [z, w1b, z]])
    in_spec = pl.BlockSpec((d, _T), lambda i: (0, i))
    full = lambda a: pl.BlockSpec(a.shape, lambda i: (0,) * a.ndim)
    out_spec = pl.BlockSpec((_T, 4 * d), lambda i: (i, 0))
    out_sh = jax.ShapeDtypeStruct((vp, 4 * d), jnp.float32)
    return pl.pallas_call(
        _pack_body,
        grid=(g,),
        in_specs=[in_spec] * 4 + [full(ru), full(ri)],
        out_specs=(out_spec, out_spec),
        out_shape=(out_sh, out_sh),
        compiler_params=pltpu.CompilerParams(
            fuse_transposed_lhs_in_matmul=True),
    )(ugT, umT, igT, imT, ru, ri)


def _sc_gather_body(uid_hbm, iid_hbm, upack, ipack, uout, iout,
                    ids_v, rows_v, sem, *, rows_per_w, bpw):
    wid = lax.axis_index("s") * _NC + lax.axis_index("c")
    rbase = wid * rows_per_w
    base = wid * bpw
    for (src_ids, tab, out) in ((uid_hbm, upack, uout),
                                (iid_hbm, ipack, iout)):
        pltpu.sync_copy(src_ids.at[pl.ds(rbase, rows_per_w)], ids_v)
        handles = []
        for j in range(rows_per_w):
            sl = pl.ds(j * _CHUNK, _CHUNK)
            handles.append(
                pltpu.async_copy(tab.at[ids_v.at[j]], rows_v.at[sl], sem))
        for h in handles:
            h.wait()
        pltpu.sync_copy(rows_v, out.at[pl.ds(base, bpw)])


@functools.partial(jax.jit, static_argnames=("b",))
def _sc_gather(uid2d, iid2d, upack, ipack, *, b):
    bpw = b // _NW
    rows_per_w = bpw // _CHUNK
    w = upack.shape[1]
    mesh = plsc.VectorSubcoreMesh(core_axis_name="c", subcore_axis_name="s")
    out_sh = jax.ShapeDtypeStruct((b, w), jnp.float32)
    f = pl.kernel(
        functools.partial(_sc_gather_body, rows_per_w=rows_per_w, bpw=bpw),
        mesh=mesh,
        out_type=(out_sh, out_sh),
        scratch_types=[
            pltpu.VMEM((rows_per_w, _CHUNK), jnp.int32),
            pltpu.VMEM((bpw, w), jnp.float32),
            pltpu.SemaphoreType.DMA,
        ],
    )
    return f(uid2d, iid2d, upack, ipack)


def _tc_mlp_body(usel, isel, b1, w2, b2, wfa, wfb, bf, out):
    d = wfa.shape[0]
    ug = usel[:, :d]
    ig = isel[:, :d]
    h = usel[:, d:3 * d] + isel[:, d:3 * d] + b1[...]
    h = jnp.maximum(h, 0.0)
    h = jnp.maximum(h @ w2[...] + b2[...], 0.0)
    out[...] = (ug * ig) @ wfa[...] + h @ wfb[...] + bf[...]


@functools.partial(jax.jit, static_argnames=("blk",))
def _tc_mlp(usel, isel, b1, w2, b2, wfa, wfb, bf, *, blk):
    b, w = usel.shape
    grid = (b // blk,)
    row_spec = pl.BlockSpec((blk, w), lambda i: (i, 0))
    full = lambda a: pl.BlockSpec(a.shape, lambda i: (0,) * a.ndim)
    return pl.pallas_call(
        _tc_mlp_body,
        grid=grid,
        in_specs=[row_spec, row_spec,
                  full(b1), full(w2), full(b2),
                  full(wfa), full(wfb), full(bf)],
        out_specs=pl.BlockSpec((blk, 1), lambda i: (i, 0)),
        out_shape=jax.ShapeDtypeStruct((b, 1), jnp.float32),
    )(usel, isel, b1, w2, b2, wfa, wfb, bf)


def kernel(inputs, user_emb_gmf, item_emb_gmf, user_emb_mlp, item_emb_mlp,
           W1, b1, W2, b2, Wf, bf):
    b = inputs.shape[0]
    d = user_emb_gmf.shape[1]
    uid2d = inputs[:, 0].reshape(-1, _CHUNK)
    iid2d = inputs[:, 1].reshape(-1, _CHUNK)
    upack, ipack = _tc_pack(user_emb_gmf.T, user_emb_mlp.T,
                            item_emb_gmf.T, item_emb_mlp.T,
                            W1[:d], W1[d:])
    usel, isel = _sc_gather(uid2d, iid2d, upack, ipack, b=b)
    out = _tc_mlp(usel, isel,
                  b1.reshape(1, -1),
                  W2, b2.reshape(1, -1),
                  Wf[:d], Wf[d:], bf.reshape(1, 1),
                  blk=8192)
    return out


# speedup vs baseline: 2.4075x; 1.0346x over previous
import functools

import jax
import jax.numpy as jnp
from jax import lax
from jax.experimental import pallas as pl
from jax.experimental.pallas import tpu as pltpu
from jax.experimental.pallas import tpu_sc as plsc

_INFO = plsc.get_sparse_core_info()
_NC = _INFO.num_cores
_NS = _INFO.num_subcores
_NW = _NC * _NS
_CHUNK = 128
_T = 8192


def _pack_body(ugT, umT, igT, imT, ru, ri, upack, ipack):
    dims = (((0,), (0,)), ((), ()))
    xu = jnp.concatenate([ugT[...], umT[...]], axis=0)
    xi = jnp.concatenate([igT[...], imT[...]], axis=0)
    upack[...] = lax.dot_general(xu, ru[...], dims,
                                 preferred_element_type=jnp.float32)
    ipack[...] = lax.dot_general(xi, ri[...], dims,
                                 preferred_element_type=jnp.float32)


@functools.partial(jax.jit, static_argnames=())
def _tc_pack(ugT, umT, igT, imT, w1a, w1b):
    d, v = ugT.shape
    g = (v + _T - 1) // _T
    vp = g * _T
    eye = jnp.eye(d, dtype=jnp.float32)
    z = jnp.zeros((d, d), jnp.float32)
    zwide = jnp.zeros((d, 2 * d), jnp.float32)
    ru = jnp.block([[eye, zwide, z], [z, w1a, z]])
    ri = jnp.block([[eye, zwide, z], [z, w1b, z]])
    in_spec = pl.BlockSpec((d, _T), lambda i: (0, i))
    full = lambda a: pl.BlockSpec(a.shape, lambda i: (0,) * a.ndim)
    out_spec = pl.BlockSpec((_T, 4 * d), lambda i: (i, 0))
    out_sh = jax.ShapeDtypeStruct((vp, 4 * d), jnp.float32)
    return pl.pallas_call(
        _pack_body,
        grid=(g,),
        in_specs=[in_spec] * 4 + [full(ru), full(ri)],
        out_specs=(out_spec, out_spec),
        out_shape=(out_sh, out_sh),
        compiler_params=pltpu.CompilerParams(
            fuse_transposed_lhs_in_matmul=True),
    )(ugT, umT, igT, imT, ru, ri)


def _sc_gather_body(uid_hbm, iid_hbm, upack, ipack, uout, iout,
                    ids_v, rows_v, sem, *, rows_per_w, bpw):
    wid = lax.axis_index("s") * _NC + lax.axis_index("c")
    rbase = wid * rows_per_w
    base = wid * bpw
    for (src_ids, tab, out) in ((uid_hbm, upack, uout),
                                (iid_hbm, ipack, iout)):
        pltpu.sync_copy(src_ids.at[pl.ds(rbase, rows_per_w)], ids_v)
        handles = []
        for j in range(rows_per_w):
            sl = pl.ds(j * _CHUNK, _CHUNK)
            handles.append(
                pltpu.async_copy(tab.at[ids_v.at[j]], rows_v.at[sl], sem))
        for h in handles:
            h.wait()
        pltpu.sync_copy(rows_v, out.at[pl.ds(base, bpw)])


@functools.partial(jax.jit, static_argnames=("b",))
def _sc_gather(uid2d, iid2d, upack, ipack, *, b):
    bpw = b // _NW
    rows_per_w = bpw // _CHUNK
    w = upack.shape[1]
    mesh = plsc.VectorSubcoreMesh(core_axis_name="c", subcore_axis_name="s")
    out_sh = jax.ShapeDtypeStruct((b, w), jnp.float32)
    f = pl.kernel(
        functools.partial(_sc_gather_body, rows_per_w=rows_per_w, bpw=bpw),
        mesh=mesh,
        out_type=(out_sh, out_sh),
        scratch_types=[
            pltpu.VMEM((rows_per_w, _CHUNK), jnp.int32),
            pltpu.VMEM((bpw, w), jnp.float32),
            pltpu.SemaphoreType.DMA,
        ],
    )
    return f(uid2d, iid2d, upack, ipack)


def _tc_mlp_body(usel, isel, b1, w2, b2, wfa, wfb, bf, out):
    d = wfa.shape[0]
    ug = usel[:, :d]
    ig = isel[:, :d]
    h = usel[:, d:3 * d] + isel[:, d:3 * d] + b1[...]
    h = jnp.maximum(h, 0.0)
    h = jnp.maximum(h @ w2[...] + b2[...], 0.0)
    out[...] = (ug * ig) @ wfa[...] + h @ wfb[...] + bf[...]


@functools.partial(jax.jit, static_argnames=("blk",))
def _tc_mlp(usel, isel, b1, w2, b2, wfa, wfb, bf, *, blk):
    b, w = usel.shape
    grid = (b // blk,)
    row_spec = pl.BlockSpec((blk, w), lambda i: (i, 0))
    full = lambda a: pl.BlockSpec(a.shape, lambda i: (0,) * a.ndim)
    return pl.pallas_call(
        _tc_mlp_body,
        grid=grid,
        in_specs=[row_spec, row_spec,
                  full(b1), full(w2), full(b2),
                  full(wfa), full(wfb), full(bf)],
        out_specs=pl.BlockSpec((blk, 1), lambda i: (i, 0)),
        out_shape=jax.ShapeDtypeStruct((b, 1), jnp.float32),
    )(usel, isel, b1, w2, b2, wfa, wfb, bf)


def kernel(inputs, user_emb_gmf, item_emb_gmf, user_emb_mlp, item_emb_mlp,
           W1, b1, W2, b2, Wf, bf):
    b = inputs.shape[0]
    d = user_emb_gmf.shape[1]
    uid2d = inputs[:, 0].reshape(-1, _CHUNK)
    iid2d = inputs[:, 1].reshape(-1, _CHUNK)
    upack, ipack = _tc_pack(user_emb_gmf.T, user_emb_mlp.T,
                            item_emb_gmf.T, item_emb_mlp.T,
                            W1[:d], W1[d:])
    usel, isel = _sc_gather(uid2d, iid2d, upack, ipack, b=b)
    out = _tc_mlp(usel, isel,
                  b1.reshape(1, -1),
                  W2, b2.reshape(1, -1),
                  Wf[:d], Wf[d:], bf.reshape(1, 1),
                  blk=8192)
    return out
